# baseline (speedup 1.0000x reference)
import functools

import jax
import jax.numpy as jnp
from jax.scipy.special import gammaln
from jax.experimental import pallas as pl
from jax.experimental.pallas import tpu as pltpu

_C = 20
_VAR = 1.0
_EPS = 1e-16
_W = 640
_G = _W // _C


def _soft_table():
    n = jnp.float32(_C - 1)
    ks = jnp.arange(_C, dtype=jnp.float32)
    ps = ks / n
    eps = jnp.float32(1e-5)
    zero = jnp.float32(0.0)
    mu = ks
    alpha = jnp.sqrt(jnp.maximum(mu * (1.0 - ps) - _VAR, zero)
                     / (jnp.maximum(mu, eps) * (1.0 + mu / jnp.maximum(n - mu, eps))))
    mu_p = mu[:, None, None]
    ks_p = ks[None, :, None]
    i_p = ks[None, None, :]
    ps2 = jnp.stack([ps + alpha, ps - mu * alpha / jnp.maximum(n - mu, eps)], axis=0)
    valid = jnp.logical_and(i_p <= mu_p, i_p >= mu_p + ks_p - n)
    validf = valid.astype(jnp.float32)
    binomials = jnp.exp(
        gammaln(n - mu_p + 1.0) + gammaln(mu_p + 1.0)
        - gammaln(jnp.maximum(ks_p - i_p + 1.0, 1.0))
        - gammaln(i_p + 1.0)
        - gammaln(jnp.maximum(mu_p - i_p + 1.0, 1.0))
        - gammaln(jnp.maximum(n - mu_p - ks_p + i_p + 1.0, 1.0))
    ) * validf
    p = ps2[:, :, None, None]
    stable = jnp.logical_not(jnp.logical_or(jnp.isclose(p, 0.0), jnp.isclose(p, 1.0)))
    sn = stable.astype(jnp.float32)
    p = jnp.where(stable, p, 0.5)
    products = jnp.exp(
        (jnp.log(p[0]) * i_p
         + jnp.log(1.0 - p[0]) * (mu_p - i_p)
         + jnp.log(p[1]) * (ks_p - i_p) * sn[0]
         + jnp.log(1.0 - p[1]) * (n - mu_p - ks_p + i_p))
        * sn[1] * validf
    )
    return (binomials * products).sum(axis=-1)


def _body(x_ref, t_ref, e_ref, swt_ref, lm_ref, o_ref, acc_ref, *, inv_b):
    i = pl.program_id(0)

    @pl.when(i == 0)
    def _init():
        acc_ref[0, 0] = jnp.float32(0.0)

    x = x_ref[...]
    t = t_ref[...].astype(jnp.float32)
    logx = jnp.log(x + jnp.float32(_EPS))
    t640 = jax.lax.dot_general(
        t, e_ref[...], (((1,), (0,)), ((), ())),
        preferred_element_type=jnp.float32)
    cand = jax.lax.dot_general(
        logx, swt_ref[...], (((1,), (0,)), ((), ())),
        preferred_element_type=jnp.float32)
    lane_mod = lm_ref[...][0:1, :]
    acc_ref[0, 0] += jnp.sum(jnp.where(lane_mod == t640, cand, 0.0))

    @pl.when(i == pl.num_programs(0) - 1)
    def _fin():
        o_ref[0, 0] = -acc_ref[0, 0] * jnp.float32(inv_b) - jnp.float32(1.0)


def kernel(inputs, targets):
    b = inputs.shape[0]
    rows = b * _C // _W
    rb = 1024
    while rows % rb:
        rb //= 2
    grid = rows // rb
    sw = _soft_table()
    lane = jnp.arange(_W)
    expand = (lane[None, :] // _C == jnp.arange(_G)[:, None]).astype(jnp.float32)
    swt = jnp.where(lane[:, None] // _C == lane[None, :] // _C,
                    sw[lane[None, :] % _C, lane[:, None] % _C],
                    jnp.float32(0.0))
    lane_mod = jnp.broadcast_to((lane % _C).astype(jnp.float32)[None, :], (8, _W))
    x640 = inputs.reshape(rows, _W)
    t32 = targets.astype(jnp.int32).reshape(rows, _G)
    out = pl.pallas_call(
        functools.partial(_body, inv_b=1.0 / b),
        grid=(grid,),
        in_specs=[
            pl.BlockSpec((rb, _W), lambda i: (i, 0)),
            pl.BlockSpec((rb, _G), lambda i: (i, 0)),
            pl.BlockSpec((_G, _W), lambda i: (0, 0)),
            pl.BlockSpec((_W, _W), lambda i: (0, 0)),
            pl.BlockSpec((8, _W), lambda i: (0, 0)),
        ],
        out_specs=pl.BlockSpec(memory_space=pltpu.SMEM),
        out_shape=jax.ShapeDtypeStruct((1, 1), jnp.float32),
        scratch_shapes=[pltpu.SMEM((1, 1), jnp.float32)],
        compiler_params=pltpu.CompilerParams(
            dimension_semantics=("arbitrary",)),
    )(x640, t32, expand, swt, lane_mod)
    return out[0, 0]


# speedup vs baseline: 4.8765x; 4.8765x over previous
import functools

import jax
import jax.numpy as jnp
from jax.scipy.special import gammaln
from jax.experimental import pallas as pl
from jax.experimental.pallas import tpu as pltpu

_C = 20
_VAR = 1.0
_EPS = 1e-16


def _soft_table():
    n = jnp.float32(_C - 1)
    ks = jnp.arange(_C, dtype=jnp.float32)
    ps = ks / n
    eps = jnp.float32(1e-5)
    zero = jnp.float32(0.0)
    mu = ks
    alpha = jnp.sqrt(jnp.maximum(mu * (1.0 - ps) - _VAR, zero)
                     / (jnp.maximum(mu, eps) * (1.0 + mu / jnp.maximum(n - mu, eps))))
    mu_p = mu[:, None, None]
    ks_p = ks[None, :, None]
    i_p = ks[None, None, :]
    ps2 = jnp.stack([ps + alpha, ps - mu * alpha / jnp.maximum(n - mu, eps)], axis=0)
    valid = jnp.logical_and(i_p <= mu_p, i_p >= mu_p + ks_p - n)
    validf = valid.astype(jnp.float32)
    binomials = jnp.exp(
        gammaln(n - mu_p + 1.0) + gammaln(mu_p + 1.0)
        - gammaln(jnp.maximum(ks_p - i_p + 1.0, 1.0))
        - gammaln(i_p + 1.0)
        - gammaln(jnp.maximum(mu_p - i_p + 1.0, 1.0))
        - gammaln(jnp.maximum(n - mu_p - ks_p + i_p + 1.0, 1.0))
    ) * validf
    p = ps2[:, :, None, None]
    stable = jnp.logical_not(jnp.logical_or(jnp.isclose(p, 0.0), jnp.isclose(p, 1.0)))
    sn = stable.astype(jnp.float32)
    p = jnp.where(stable, p, 0.5)
    products = jnp.exp(
        (jnp.log(p[0]) * i_p
         + jnp.log(1.0 - p[0]) * (mu_p - i_p)
         + jnp.log(p[1]) * (ks_p - i_p) * sn[0]
         + jnp.log(1.0 - p[1]) * (n - mu_p - ks_p + i_p))
        * sn[1] * validf
    )
    return (binomials * products).sum(axis=-1)


def _body(x_ref, t_ref, sw_ref, o_ref, acc_ref, *, inv_b):
    i = pl.program_id(0)

    @pl.when(i == 0)
    def _init():
        acc_ref[0, 0] = jnp.float32(0.0)

    x = x_ref[...]
    t = t_ref[...]
    logx = jnp.log(x + jnp.float32(_EPS))
    onehot = (jax.lax.broadcasted_iota(jnp.int32, (t.shape[0], 32), 1)
              == t).astype(jnp.float32)
    gath = jax.lax.dot_general(
        onehot, sw_ref[...], (((1,), (0,)), ((), ())),
        preferred_element_type=jnp.float32)
    acc_ref[0, 0] += jnp.sum(logx * gath[:, :_C])

    @pl.when(i == pl.num_programs(0) - 1)
    def _fin():
        o_ref[0, 0] = -acc_ref[0, 0] * jnp.float32(inv_b) - jnp.float32(1.0)


def kernel(inputs, targets):
    b = inputs.shape[0]
    blk = 8192
    grid = b // blk
    sw = _soft_table()
    swpad = jnp.zeros((32, 128), jnp.float32).at[:_C, :_C].set(sw)
    t2 = targets.astype(jnp.int32).reshape(b, 1)
    out = pl.pallas_call(
        functools.partial(_body, inv_b=1.0 / b),
        grid=(grid,),
        in_specs=[
            pl.BlockSpec((blk, _C), lambda i: (i, 0)),
            pl.BlockSpec((blk, 1), lambda i: (i, 0)),
            pl.BlockSpec((32, 128), lambda i: (0, 0)),
        ],
        out_specs=pl.BlockSpec(memory_space=pltpu.SMEM),
        out_shape=jax.ShapeDtypeStruct((1, 1), jnp.float32),
        scratch_shapes=[pltpu.SMEM((1, 1), jnp.float32)],
        compiler_params=pltpu.CompilerParams(
            dimension_semantics=("arbitrary",)),
    )(inputs, t2, swpad)
    return out[0, 0]


# speedup vs baseline: 8.3910x; 1.7207x over previous
import functools

import jax
import jax.numpy as jnp
from jax.scipy.special import gammaln
from jax.experimental import pallas as pl
from jax.experimental.pallas import tpu as pltpu

_C = 20
_VAR = 1.0
_EPS = 1e-16


def _soft_table():
    n = jnp.float32(_C - 1)
    ks = jnp.arange(_C, dtype=jnp.float32)
    ps = ks / n
    eps = jnp.float32(1e-5)
    zero = jnp.float32(0.0)
    mu = ks
    alpha = jnp.sqrt(jnp.maximum(mu * (1.0 - ps) - _VAR, zero)
                     / (jnp.maximum(mu, eps) * (1.0 + mu / jnp.maximum(n - mu, eps))))
    mu_p = mu[:, None, None]
    ks_p = ks[None, :, None]
    i_p = ks[None, None, :]
    ps2 = jnp.stack([ps + alpha, ps - mu * alpha / jnp.maximum(n - mu, eps)], axis=0)
    valid = jnp.logical_and(i_p <= mu_p, i_p >= mu_p + ks_p - n)
    validf = valid.astype(jnp.float32)
    binomials = jnp.exp(
        gammaln(n - mu_p + 1.0) + gammaln(mu_p + 1.0)
        - gammaln(jnp.maximum(ks_p - i_p + 1.0, 1.0))
        - gammaln(i_p + 1.0)
        - gammaln(jnp.maximum(mu_p - i_p + 1.0, 1.0))
        - gammaln(jnp.maximum(n - mu_p - ks_p + i_p + 1.0, 1.0))
    ) * validf
    p = ps2[:, :, None, None]
    stable = jnp.logical_not(jnp.logical_or(jnp.isclose(p, 0.0), jnp.isclose(p, 1.0)))
    sn = stable.astype(jnp.float32)
    p = jnp.where(stable, p, 0.5)
    products = jnp.exp(
        (jnp.log(p[0]) * i_p
         + jnp.log(1.0 - p[0]) * (mu_p - i_p)
         + jnp.log(p[1]) * (ks_p - i_p) * sn[0]
         + jnp.log(1.0 - p[1]) * (n - mu_p - ks_p + i_p))
        * sn[1] * validf
    )
    return (binomials * products).sum(axis=-1)


def _body(x_ref, t_ref, sw_ref, o_ref, acc_ref, *, inv_b):
    i = pl.program_id(0)

    @pl.when(i == 0)
    def _init():
        acc_ref[...] = jnp.zeros_like(acc_ref)

    x = x_ref[...]
    t = t_ref[0]
    blk = x.shape[0]
    logx = jnp.log(x + jnp.float32(_EPS))
    ot = (jax.lax.broadcasted_iota(jnp.int32, (32, blk), 0)
          == t).astype(jnp.float32)
    m = jax.lax.dot_general(
        ot, logx, (((1,), (0,)), ((), ())),
        preferred_element_type=jnp.float32)
    acc_ref[...] += m

    @pl.when(i == pl.num_programs(0) - 1)
    def _fin():
        o_ref[0, 0] = (-jnp.sum(acc_ref[...] * sw_ref[...]) * jnp.float32(inv_b)
                       - jnp.float32(1.0))


def kernel(inputs, targets):
    b = inputs.shape[0]
    blk = 8192
    grid = b // blk
    sw = _soft_table()
    swpad = jnp.zeros((32, _C), jnp.float32).at[:_C, :].set(sw)
    t3 = targets.astype(jnp.int32).reshape(grid, 1, blk)
    out = pl.pallas_call(
        functools.partial(_body, inv_b=1.0 / b),
        grid=(grid,),
        in_specs=[
            pl.BlockSpec((blk, _C), lambda i: (i, 0)),
            pl.BlockSpec((1, 1, blk), lambda i: (i, 0, 0)),
            pl.BlockSpec((32, _C), lambda i: (0, 0)),
        ],
        out_specs=pl.BlockSpec(memory_space=pltpu.SMEM),
        out_shape=jax.ShapeDtypeStruct((1, 1), jnp.float32),
        scratch_shapes=[pltpu.VMEM((32, _C), jnp.float32)],
        compiler_params=pltpu.CompilerParams(
            dimension_semantics=("arbitrary",)),
    )(inputs, t3, swpad)
    return out[0, 0]
